# baseline (speedup 1.0000x reference)
import functools

import jax
import jax.numpy as jnp
from jax import lax
from jax.experimental import pallas as pl
from jax.experimental.pallas import tpu as pltpu
from jax.experimental.pallas import tpu_sc as plsc

_B = 16384
_D = 64
_NNEG = 5
_V = 1000000

_info = plsc.get_sparse_core_info()
_NC = _info.num_cores
_NS = _info.num_subcores
_NW = _NC * _NS

_PC_PER_W = _B // _NW
_NEG_PER_W = _B * _NNEG // _NW
_CHUNK = 256


def _section(table3, idx_v, idx_off, out_hbm, out_base, n_chunks,
             rows_v, sem):
    def chunk(c, _):
        coff = idx_off + c * _CHUNK
        copies = []
        for k in range(_CHUNK):
            if k % 16 == 0:
                iv = idx_v[pl.ds(coff + k, 16)]
            i = iv[k % 16]
            copies.append(pltpu.async_copy(
                table3.at[i >> 3, i & 7], rows_v.at[k], sem))
        for h in copies:
            h.wait()
        pltpu.sync_copy(rows_v, out_hbm.at[pl.ds(out_base + c * _CHUNK, _CHUNK)])
        return _

    lax.fori_loop(0, n_chunks, chunk, 0)


def _sc_body(center_hbm, context_hbm, pc_idx_hbm, pctx_idx_hbm, neg_idx_hbm,
             pc_out, pctx_out, neg_out,
             idx_v, rows_v, sem):
    center3 = center_hbm.reshape(_V // 8, 8, _D)
    context3 = context_hbm.reshape(_V // 8, 8, _D)
    w = lax.axis_index("s") * _NC + lax.axis_index("c")
    base = w * _PC_PER_W
    nbase = w * _NEG_PER_W

    pltpu.sync_copy(pc_idx_hbm.at[pl.ds(base, _PC_PER_W)],
                    idx_v.at[pl.ds(0, _PC_PER_W)])
    pltpu.sync_copy(pctx_idx_hbm.at[pl.ds(base, _PC_PER_W)],
                    idx_v.at[pl.ds(_PC_PER_W, _PC_PER_W)])
    pltpu.sync_copy(neg_idx_hbm.at[pl.ds(nbase, _NEG_PER_W)],
                    idx_v.at[pl.ds(2 * _PC_PER_W, _NEG_PER_W)])

    _section(center3, idx_v, 0, pc_out, base, _PC_PER_W // _CHUNK, rows_v, sem)
    _section(context3, idx_v, _PC_PER_W, pctx_out, base,
             _PC_PER_W // _CHUNK, rows_v, sem)
    _section(context3, idx_v, 2 * _PC_PER_W, neg_out, nbase,
             _NEG_PER_W // _CHUNK, rows_v, sem)


@jax.jit
def _fasttext_gather(center_W, context_W, pc_idx, pctx_idx, neg_idx):
    mesh = plsc.VectorSubcoreMesh(core_axis_name="c", subcore_axis_name="s")
    return pl.kernel(
        _sc_body,
        mesh=mesh,
        out_type=(
            jax.ShapeDtypeStruct((_B, _D), jnp.float32),
            jax.ShapeDtypeStruct((_B, _D), jnp.float32),
            jax.ShapeDtypeStruct((_B * _NNEG, _D), jnp.float32),
        ),
        scratch_types=[
            pltpu.VMEM((2 * _PC_PER_W + _NEG_PER_W,), jnp.int32),
            pltpu.VMEM((_CHUNK, _D), jnp.float32),
            pltpu.SemaphoreType.DMA,
        ],
    )(center_W, context_W, pc_idx, pctx_idx, neg_idx)


def kernel(center_W, context_W, pos_center, pos_context, neg_context):
    pc_idx = pos_center.astype(jnp.int32)
    pctx_idx = pos_context.astype(jnp.int32)
    neg_idx = neg_context.reshape(-1).astype(jnp.int32)
    pc, pctx, nctx = _fasttext_gather(center_W, context_W, pc_idx, pctx_idx, neg_idx)
    return pc, pctx, nctx.reshape(_B, _NNEG, _D)


# speedup vs baseline: 1.4668x; 1.4668x over previous
import functools

import jax
import jax.numpy as jnp
from jax import lax
from jax.experimental import pallas as pl
from jax.experimental.pallas import tpu as pltpu
from jax.experimental.pallas import tpu_sc as plsc

_B = 16384
_D = 64
_NNEG = 5
_V = 1000000

_info = plsc.get_sparse_core_info()
_NC = _info.num_cores
_NS = _info.num_subcores
_NW = _NC * _NS

_PC_PER_W = _B // _NW
_NEG_PER_W = _B * _NNEG // _NW
_CHUNK = 256


def _section(table3, idx_v, idx_off, out_hbm, out_base, n_chunks,
             rows_v, sem):
    def chunk(c, carry):
        coff = idx_off + c * _CHUNK
        copies = []
        for k in range(_CHUNK):
            if k % 16 == 0:
                iv = idx_v[pl.ds(coff + k, 16)]
            i = iv[k % 16]
            copies.append(pltpu.async_copy(
                table3.at[i >> 3, i & 7], rows_v.at[k], sem))
        for h in copies:
            h.wait()
        pltpu.sync_copy(rows_v, out_hbm.at[pl.ds(out_base + c * _CHUNK, _CHUNK)])
        return carry

    lax.fori_loop(0, n_chunks, chunk, 0)


def _neg_section(table3, idx_v, idx_off, out3, item_base, n_chunks,
                 rows_v3, sem):
    def chunk(c, carry):
        coff = idx_off + c * (_NNEG * 64)
        copies = []
        for k in range(_NNEG * 64):
            if k % 16 == 0:
                iv = idx_v[pl.ds(coff + k, 16)]
            i = iv[k % 16]
            copies.append(pltpu.async_copy(
                table3.at[i >> 3, i & 7], rows_v3.at[k // _NNEG, k % _NNEG],
                sem))
        for h in copies:
            h.wait()
        pltpu.sync_copy(rows_v3, out3.at[pl.ds(item_base + c * 64, 64)])
        return carry

    lax.fori_loop(0, n_chunks, chunk, 0)


def _sc_body(center3, context3, pc_idx_hbm, pctx_idx_hbm, neg_idx_hbm,
             pc_out, pctx_out, neg_out3,
             idx_v, rows_v, rows_v3, sem):
    w = lax.axis_index("s") * _NC + lax.axis_index("c")
    base = w * _PC_PER_W
    nbase = w * _NEG_PER_W

    pltpu.sync_copy(pc_idx_hbm.at[pl.ds(base, _PC_PER_W)],
                    idx_v.at[pl.ds(0, _PC_PER_W)])
    pltpu.sync_copy(pctx_idx_hbm.at[pl.ds(base, _PC_PER_W)],
                    idx_v.at[pl.ds(_PC_PER_W, _PC_PER_W)])
    pltpu.sync_copy(neg_idx_hbm.at[pl.ds(nbase, _NEG_PER_W)],
                    idx_v.at[pl.ds(2 * _PC_PER_W, _NEG_PER_W)])

    _section(center3, idx_v, 0, pc_out, base, _PC_PER_W // _CHUNK, rows_v, sem)
    _section(context3, idx_v, _PC_PER_W, pctx_out, base,
             _PC_PER_W // _CHUNK, rows_v, sem)
    _neg_section(context3, idx_v, 2 * _PC_PER_W, neg_out3, w * _PC_PER_W,
                 _PC_PER_W // 64, rows_v3, sem)


@jax.jit
def _fasttext_gather(center_W, context_W, pc_idx, pctx_idx, neg_idx):
    center3 = center_W.reshape(_V // 8, 8, _D)
    context3 = context_W.reshape(_V // 8, 8, _D)
    mesh = plsc.VectorSubcoreMesh(core_axis_name="c", subcore_axis_name="s")
    return pl.kernel(
        _sc_body,
        mesh=mesh,
        out_type=(
            jax.ShapeDtypeStruct((_B, _D), jnp.float32),
            jax.ShapeDtypeStruct((_B, _D), jnp.float32),
            jax.ShapeDtypeStruct((_B, _NNEG, _D), jnp.float32),
        ),
        scratch_types=[
            pltpu.VMEM((2 * _PC_PER_W + _NEG_PER_W,), jnp.int32),
            pltpu.VMEM((_CHUNK, _D), jnp.float32),
            pltpu.VMEM((64, _NNEG, _D), jnp.float32),
            pltpu.SemaphoreType.DMA,
        ],
    )(center3, context3, pc_idx, pctx_idx, neg_idx)


def kernel(center_W, context_W, pos_center, pos_context, neg_context):
    pc_idx = pos_center.astype(jnp.int32)
    pctx_idx = pos_context.astype(jnp.int32)
    neg_idx = neg_context.reshape(-1).astype(jnp.int32)
    return _fasttext_gather(center_W, context_W, pc_idx, pctx_idx, neg_idx)
